# Initial kernel scaffold; baseline (speedup 1.0000x reference)
import jax
import jax.numpy as jnp
from jax.experimental import pallas as pl


def kernel(x, edge_index, subgraph_node_idx, W_root0, W_nbr0, b0, gamma0, beta0, W_root1, W_nbr1, b1, gamma1, beta1, W_root2, W_nbr2, b2, gamma2, beta2, Wf1, bf1, Wf2, bf2):
    raise NotImplementedError("write your pallas kernel here")


# speedup vs baseline: 4.1699x; 4.1699x over previous
import functools

import jax
import jax.numpy as jnp
from jax import lax
from jax.experimental import pallas as pl
from jax.experimental.pallas import tpu as pltpu
from jax.experimental.pallas import tpu_sc as plsc

N = 10000
E = 320000
D = 128
T = 10

NC = 2
NS = 16
NW = NC * NS
N_TAB = 10112
ROWS_PER_TILE = N_TAB // NS
CH = 80

E_PER_W = E // NW
N_CHUNKS = E_PER_W // CH

R_PAD = 10240
R_PER_W = R_PAD // NW
R_CHUNKS = R_PER_W // CH

_MESH = plsc.VectorSubcoreMesh(core_axis_name="c", subcore_axis_name="s")


def _zero_acc(zeros_hbm, acc_sh, tid):
    pltpu.sync_copy(
        zeros_hbm.at[pl.ds(tid * ROWS_PER_TILE, ROWS_PER_TILE)],
        acc_sh.at[pl.ds(tid * ROWS_PER_TILE, ROWS_PER_TILE)],
    )


def _drain_acc(acc_sh, out_hbm, cid, tid):
    pltpu.sync_copy(
        acc_sh.at[pl.ds(tid * ROWS_PER_TILE, ROWS_PER_TILE)],
        out_hbm.at[cid, pl.ds(tid * ROWS_PER_TILE, ROWS_PER_TILE)],
    )


def _make_sc_scatter(e_per_w, n_chunks):
    @functools.partial(
        pl.kernel,
        out_type=jax.ShapeDtypeStruct((NC, N_TAB, D), jnp.float32),
        mesh=_MESH,
        scratch_types=[
            pltpu.VMEM((CH,), jnp.int32),
            pltpu.VMEM((1, CH), jnp.int32),
            pltpu.VMEM((CH, D), jnp.float32),
            pltpu.MemorySpace.VMEM_SHARED((N_TAB, D), jnp.float32),
            pltpu.SemaphoreType.DMA,
        ],
    )
    def sc_scatter(table_hbm, src_hbm, dst_hbm, zeros_hbm, out_hbm,
                   sidx_v, didx_v, rows_v, acc_sh, sem):
        cid = lax.axis_index("c")
        tid = lax.axis_index("s")
        wid = cid * NS + tid
        _zero_acc(zeros_hbm, acc_sh, tid)
        plsc.subcore_barrier()
        base0 = wid * e_per_w

        @pl.loop(0, n_chunks)
        def _chunks(i):
            base = base0 + i * CH
            pltpu.sync_copy(src_hbm.at[pl.ds(base, CH)], sidx_v)
            pltpu.sync_copy(dst_hbm.at[pl.ds(base, CH)], didx_v.at[0])
            pltpu.async_copy(table_hbm.at[sidx_v], rows_v, sem).wait()
            pltpu.sync_copy(rows_v, acc_sh.at[didx_v.at[0]], add=True)

        plsc.subcore_barrier()
        _drain_acc(acc_sh, out_hbm, cid, tid)

    return sc_scatter


_sc_scatter_edges = _make_sc_scatter(E_PER_W, N_CHUNKS)
_sc_scatter_read = _make_sc_scatter(R_PER_W, R_CHUNKS)


@functools.partial(
    pl.kernel,
    out_type=(
        jax.ShapeDtypeStruct((NC, N_TAB, D), jnp.float32),
        jax.ShapeDtypeStruct((NC, N_TAB, D), jnp.float32),
    ),
    mesh=_MESH,
    scratch_types=[
        pltpu.VMEM((1, CH), jnp.int32),
        pltpu.VMEM((CH, D), jnp.float32),
        pltpu.MemorySpace.VMEM_SHARED((N_TAB, D), jnp.float32),
    ],
)
def _sc_counts(dst_hbm, sgi_hbm, zeros_hbm, ones_hbm, cnt_out, den_out,
               didx_v, ones_v, acc_sh):
    cid = lax.axis_index("c")
    tid = lax.axis_index("s")
    wid = cid * NS + tid
    pltpu.sync_copy(ones_hbm, ones_v)

    _zero_acc(zeros_hbm, acc_sh, tid)
    plsc.subcore_barrier()
    base0 = wid * E_PER_W

    @pl.loop(0, N_CHUNKS)
    def _edges(i):
        pltpu.sync_copy(dst_hbm.at[pl.ds(base0 + i * CH, CH)], didx_v.at[0])
        pltpu.sync_copy(ones_v, acc_sh.at[didx_v.at[0]], add=True)

    plsc.subcore_barrier()
    _drain_acc(acc_sh, cnt_out, cid, tid)
    plsc.subcore_barrier()

    _zero_acc(zeros_hbm, acc_sh, tid)
    plsc.subcore_barrier()
    base1 = wid * R_PER_W

    @pl.loop(0, R_CHUNKS)
    def _reads(i):
        pltpu.sync_copy(sgi_hbm.at[pl.ds(base1 + i * CH, CH)], didx_v.at[0])
        pltpu.sync_copy(ones_v, acc_sh.at[didx_v.at[0]], add=True)

    plsc.subcore_barrier()
    _drain_acc(acc_sh, den_out, cid, tid)


def _tc_mm(x, w):
    def body(x_ref, w_ref, o_ref):
        o_ref[...] = jnp.dot(x_ref[...], w_ref[...],
                             preferred_element_type=jnp.float32)

    return pl.pallas_call(
        body,
        out_shape=jax.ShapeDtypeStruct((x.shape[0], w.shape[1]), jnp.float32),
    )(x, w)


def _fuse_body(last, h_ref, p_ref, c_ref, wr_ref, b_ref, g_ref, be_ref,
               wn_ref, h_out, y_out=None):
    agg = p_ref[0, :N, :] + p_ref[1, :N, :]
    cnt = c_ref[0, :N, 0:1] + c_ref[1, :N, 0:1]
    u = (jnp.dot(h_ref[...], wr_ref[...], preferred_element_type=jnp.float32)
         + agg / jnp.maximum(cnt, 1.0) + b_ref[...])
    mu = jnp.mean(u, axis=0, keepdims=True)
    d = u - mu
    var = jnp.mean(d * d, axis=0, keepdims=True)
    hn = d * lax.rsqrt(var + 1e-5) * g_ref[...] + be_ref[...]
    h_new = jnp.maximum(hn, 0.0)
    if last:
        h_out[:N, :] = h_new
        h_out[N:, :] = jnp.zeros((R_PAD - N, D), jnp.float32)
    else:
        h_out[...] = h_new
        y_out[...] = jnp.dot(h_new, wn_ref[...],
                             preferred_element_type=jnp.float32)


def _tc_fuse(h, p, c, w_root, b, g, be, w_nbr_next):
    return pl.pallas_call(
        functools.partial(_fuse_body, False),
        out_shape=(
            jax.ShapeDtypeStruct((N, D), jnp.float32),
            jax.ShapeDtypeStruct((N, D), jnp.float32),
        ),
    )(h, p, c, w_root, b, g, be, w_nbr_next)


def _tc_fin(h, p, c, w_root, b, g, be):
    dummy_w = jnp.zeros((1, 1), jnp.float32)
    return pl.pallas_call(
        functools.partial(_fuse_body, True),
        out_shape=jax.ShapeDtypeStruct((R_PAD, D), jnp.float32),
    )(h, p, c, w_root, b, g, be, dummy_w)


def _tc_head(q, denp, wf1, bf1, wf2p, bf2p):
    def body(q_ref, d_ref, w1_ref, b1_ref, w2_ref, b2_ref, o_ref):
        num = q_ref[0, :N, :] + q_ref[1, :N, :]
        den = d_ref[0, :N, 0:1] + d_ref[1, :N, 0:1]
        xn = num / jnp.maximum(den, 1.0)
        a = jnp.maximum(
            jnp.dot(xn, w1_ref[...], preferred_element_type=jnp.float32)
            + b1_ref[...], 0.0)
        logits = (jnp.dot(a, w2_ref[...], preferred_element_type=jnp.float32)
                  + b2_ref[...])
        col = lax.broadcasted_iota(jnp.int32, (1, D), 1)
        logits = jnp.where(col < T, logits, -1e30)
        m = jnp.max(logits, axis=1, keepdims=True)
        lse = jnp.log(jnp.sum(jnp.exp(logits - m), axis=1, keepdims=True)) + m
        o_ref[...] = logits - lse

    return pl.pallas_call(
        body,
        out_shape=jax.ShapeDtypeStruct((N, D), jnp.float32),
    )(q, denp, wf1, bf1, wf2p, bf2p)


def kernel(x, edge_index, subgraph_node_idx,
           W_root0, W_nbr0, b0, gamma0, beta0,
           W_root1, W_nbr1, b1, gamma1, beta1,
           W_root2, W_nbr2, b2, gamma2, beta2,
           Wf1, bf1, Wf2, bf2):
    src = edge_index[0]
    dst = edge_index[1]
    sgi_pad = jnp.concatenate(
        [subgraph_node_idx, jnp.full((R_PAD - N,), N, jnp.int32)])
    src_lin = jnp.concatenate(
        [jnp.arange(N, dtype=jnp.int32), jnp.zeros((R_PAD - N,), jnp.int32)])
    zerosD = jnp.zeros((N_TAB, D), jnp.float32)

    onesD = jnp.ones((CH, D), jnp.float32)
    cnt_p, den_p = _sc_counts(dst, sgi_pad, zerosD, onesD)

    roots = [W_root0, W_root1, W_root2]
    nbrs = [W_nbr0, W_nbr1, W_nbr2]
    bs = [b0.reshape(1, D), b1.reshape(1, D), b2.reshape(1, D)]
    gs = [gamma0.reshape(1, D), gamma1.reshape(1, D), gamma2.reshape(1, D)]
    bes = [beta0.reshape(1, D), beta1.reshape(1, D), beta2.reshape(1, D)]

    y = _tc_mm(x, nbrs[0])
    h = x
    for i in range(3):
        p = _sc_scatter_edges(y, src, dst, zerosD)
        if i < 2:
            h, y = _tc_fuse(h, p, cnt_p, roots[i], bs[i], gs[i], bes[i],
                            nbrs[i + 1])
        else:
            h3 = _tc_fin(h, p, cnt_p, roots[i], bs[i], gs[i], bes[i])

    q = _sc_scatter_read(h3, src_lin, sgi_pad, zerosD)

    wf2p = jnp.zeros((2 * D, D), jnp.float32).at[:, :T].set(Wf2)
    bf2p = jnp.zeros((1, D), jnp.float32).at[0, :T].set(bf2)
    out = _tc_head(q, den_p, Wf1, bf1.reshape(1, 2 * D), wf2p, bf2p)
    return out[:, :T]


# speedup vs baseline: 4.9929x; 1.1974x over previous
import functools

import jax
import jax.numpy as jnp
from jax import lax
from jax.experimental import pallas as pl
from jax.experimental.pallas import tpu as pltpu
from jax.experimental.pallas import tpu_sc as plsc

N = 10000
E = 320000
D = 128
T = 10

NC = 2
NS = 16
NW = NC * NS
N_TAB = 10112
ROWS_PER_TILE = N_TAB // NS
CH = 64

E_PER_W = 10112
E_PAD = NW * E_PER_W
N_CHUNKS = E_PER_W // CH

R_PAD = 10240
R_PER_W = R_PAD // NW
R_CHUNKS = R_PER_W // CH

_MESH = plsc.VectorSubcoreMesh(core_axis_name="c", subcore_axis_name="s")


def _zero_acc(zeros_hbm, acc_sh, tid):
    pltpu.sync_copy(
        zeros_hbm.at[pl.ds(tid * ROWS_PER_TILE, ROWS_PER_TILE)],
        acc_sh.at[pl.ds(tid * ROWS_PER_TILE, ROWS_PER_TILE)],
    )


def _drain_acc(acc_sh, out_hbm, cid, tid):
    pltpu.sync_copy(
        acc_sh.at[pl.ds(tid * ROWS_PER_TILE, ROWS_PER_TILE)],
        out_hbm.at[cid, pl.ds(tid * ROWS_PER_TILE, ROWS_PER_TILE)],
    )


NR = 4
NI = 8
AHEAD_G = NR - 2
AHEAD_I = AHEAD_G + 2


def _make_sc_scatter(n_chunks):
    @functools.partial(
        pl.kernel,
        out_type=jax.ShapeDtypeStruct((NC, N_TAB, D), jnp.float32),
        mesh=_MESH,
        scratch_types=[
            [pltpu.VMEM((2, CH), jnp.int32) for _ in range(NI)],
            [pltpu.VMEM((CH, D), jnp.float32) for _ in range(NR)],
            pltpu.MemorySpace.VMEM_SHARED((N_TAB, D), jnp.float32),
            [pltpu.SemaphoreType.DMA for _ in range(NI)],
            [pltpu.SemaphoreType.DMA for _ in range(NR)],
            [pltpu.SemaphoreType.DMA for _ in range(NR)],
        ],
    )
    def sc_scatter(table_hbm, ei_hbm, zeros_hbm, out_hbm,
                   idx, rows, acc_sh, isem, gsem, ssem):
        cid = lax.axis_index("c")
        tid = lax.axis_index("s")
        wid = cid * NS + tid

        def issue_idx(j, si):
            pltpu.async_copy(ei_hbm.at[wid, j], idx[si], isem[si])

        def wait_idx(j, si):
            pltpu.make_async_copy(ei_hbm.at[wid, j], idx[si], isem[si]).wait()

        def issue_gather(j, si, sr):
            pltpu.async_copy(table_hbm.at[idx[si].at[0]], rows[sr], gsem[sr])

        def wait_gather(j, si, sr):
            pltpu.make_async_copy(table_hbm.at[idx[si].at[0]], rows[sr],
                                  gsem[sr]).wait()

        def issue_scatter(j, si, sr):
            pltpu.async_copy(rows[sr], acc_sh.at[idx[si].at[1]], ssem[sr],
                             add=True)

        def wait_scatter(j, si, sr):
            pltpu.make_async_copy(rows[sr], acc_sh.at[idx[si].at[1]],
                                  ssem[sr]).wait()

        def body(j):
            sr = j % NR
            si = j % NI
            wait_gather(j, si, sr)
            issue_scatter(j, si, sr)
            if isinstance(j, int) and j < 2:
                pass
            else:
                wait_scatter(j - 2, (j - 2) % NI, (j - 2) % NR)
            if not (isinstance(j, int) and j + AHEAD_I >= n_chunks):
                issue_idx(j + AHEAD_I, (j + AHEAD_I) % NI)
            if not (isinstance(j, int) and j + AHEAD_G >= n_chunks):
                jg = j + AHEAD_G
                wait_idx(jg, jg % NI)
                issue_gather(jg, jg % NI, jg % NR)

        for t in range(min(AHEAD_I, n_chunks)):
            issue_idx(t, t % NI)
        _zero_acc(zeros_hbm, acc_sh, tid)
        plsc.subcore_barrier()
        for t in range(min(AHEAD_G, n_chunks)):
            wait_idx(t, t % NI)
            issue_gather(t, t % NI, t % NR)

        head = min(NI, n_chunks)
        for j in range(head):
            body(j)

        n_guard = max(head, n_chunks - AHEAD_I)
        steady_end = head + ((n_guard - head) // NI) * NI
        if steady_end > head:
            @pl.loop(head, steady_end, step=NI)
            def _steady(i):
                for b in range(NI):
                    sr = b % NR
                    si = b
                    wait_gather(i + b, si, sr)
                    issue_scatter(i + b, si, sr)
                    wait_scatter(i + b - 2, (b - 2) % NI, (b - 2) % NR)
                    issue_idx(i + b + AHEAD_I, (b + AHEAD_I) % NI)
                    wait_idx(i + b + AHEAD_G, (b + AHEAD_G) % NI)
                    issue_gather(i + b + AHEAD_G, (b + AHEAD_G) % NI,
                                 (b + AHEAD_G) % NR)

        for j in range(steady_end, n_chunks):
            body(j)

        for j in range(max(0, n_chunks - 2), n_chunks):
            wait_scatter(j, j % NI, j % NR)

        plsc.subcore_barrier()
        _drain_acc(acc_sh, out_hbm, cid, tid)

    return sc_scatter


_sc_scatter_edges = _make_sc_scatter(N_CHUNKS)
_sc_scatter_read = _make_sc_scatter(R_CHUNKS)


@functools.partial(
    pl.kernel,
    out_type=(
        jax.ShapeDtypeStruct((NC, N_TAB, D), jnp.float32),
        jax.ShapeDtypeStruct((NC, N_TAB, D), jnp.float32),
    ),
    mesh=_MESH,
    scratch_types=[
        pltpu.VMEM((N_CHUNKS, CH), jnp.int32),
        pltpu.VMEM((R_CHUNKS, CH), jnp.int32),
        pltpu.VMEM((CH, D), jnp.float32),
        pltpu.MemorySpace.VMEM_SHARED((N_TAB, D), jnp.float32),
        pltpu.SemaphoreType.DMA,
    ],
)
def _sc_counts(dst_hbm, sgi_hbm, zeros_hbm, ones_hbm, cnt_out, den_out,
               didx_v, ridx_v, ones_v, acc_sh, sem):
    cid = lax.axis_index("c")
    tid = lax.axis_index("s")
    wid = cid * NS + tid
    pltpu.sync_copy(ones_hbm, ones_v)
    pltpu.sync_copy(dst_hbm.at[wid], didx_v)
    pltpu.sync_copy(sgi_hbm.at[wid], ridx_v)

    _zero_acc(zeros_hbm, acc_sh, tid)
    plsc.subcore_barrier()

    def issue(idx_ref, j):
        pltpu.async_copy(ones_v, acc_sh.at[idx_ref.at[j]], sem, add=True)

    def drain(idx_ref, j):
        pltpu.make_async_copy(ones_v, acc_sh.at[idx_ref.at[j]], sem).wait()

    W = 16

    @pl.loop(0, W)
    def _fill(j):
        issue(didx_v, j)

    @pl.loop(W, N_CHUNKS)
    def _roll(j):
        issue(didx_v, j)
        drain(didx_v, j - W)

    @pl.loop(N_CHUNKS - W, N_CHUNKS)
    def _flush(j):
        drain(didx_v, j)

    plsc.subcore_barrier()
    _drain_acc(acc_sh, cnt_out, cid, tid)
    plsc.subcore_barrier()

    _zero_acc(zeros_hbm, acc_sh, tid)
    plsc.subcore_barrier()

    for j in range(R_CHUNKS):
        issue(ridx_v, j)
    for j in range(R_CHUNKS):
        drain(ridx_v, j)

    plsc.subcore_barrier()
    _drain_acc(acc_sh, den_out, cid, tid)


def _tc_mm(x, w):
    def body(x_ref, w_ref, o_ref):
        o_ref[...] = jnp.dot(x_ref[...], w_ref[...],
                             preferred_element_type=jnp.float32)

    return pl.pallas_call(
        body,
        out_shape=jax.ShapeDtypeStruct((x.shape[0], w.shape[1]), jnp.float32),
    )(x, w)


def _fuse_body(last, h_ref, p_ref, c_ref, wr_ref, b_ref, g_ref, be_ref,
               wn_ref, h_out, y_out=None):
    agg = p_ref[0, :N, :] + p_ref[1, :N, :]
    cnt = c_ref[0, :N, 0:1] + c_ref[1, :N, 0:1]
    u = (jnp.dot(h_ref[...], wr_ref[...], preferred_element_type=jnp.float32)
         + agg / jnp.maximum(cnt, 1.0) + b_ref[...])
    mu = jnp.mean(u, axis=0, keepdims=True)
    d = u - mu
    var = jnp.mean(d * d, axis=0, keepdims=True)
    hn = d * lax.rsqrt(var + 1e-5) * g_ref[...] + be_ref[...]
    h_new = jnp.maximum(hn, 0.0)
    if last:
        h_out[:N, :] = h_new
        h_out[N:, :] = jnp.zeros((R_PAD - N, D), jnp.float32)
    else:
        h_out[...] = h_new
        y_out[...] = jnp.dot(h_new, wn_ref[...],
                             preferred_element_type=jnp.float32)


def _tc_fuse(h, p, c, w_root, b, g, be, w_nbr_next):
    return pl.pallas_call(
        functools.partial(_fuse_body, False),
        out_shape=(
            jax.ShapeDtypeStruct((N, D), jnp.float32),
            jax.ShapeDtypeStruct((N, D), jnp.float32),
        ),
    )(h, p, c, w_root, b, g, be, w_nbr_next)


def _tc_fin(h, p, c, w_root, b, g, be):
    dummy_w = jnp.zeros((1, 1), jnp.float32)
    return pl.pallas_call(
        functools.partial(_fuse_body, True),
        out_shape=jax.ShapeDtypeStruct((R_PAD, D), jnp.float32),
    )(h, p, c, w_root, b, g, be, dummy_w)


def _tc_head(q, denp, wf1, bf1, wf2p, bf2p):
    def body(q_ref, d_ref, w1_ref, b1_ref, w2_ref, b2_ref, o_ref):
        num = q_ref[0, :N, :] + q_ref[1, :N, :]
        den = d_ref[0, :N, 0:1] + d_ref[1, :N, 0:1]
        xn = num / jnp.maximum(den, 1.0)
        a = jnp.maximum(
            jnp.dot(xn, w1_ref[...], preferred_element_type=jnp.float32)
            + b1_ref[...], 0.0)
        logits = (jnp.dot(a, w2_ref[...], preferred_element_type=jnp.float32)
                  + b2_ref[...])
        col = lax.broadcasted_iota(jnp.int32, (1, D), 1)
        logits = jnp.where(col < T, logits, -1e30)
        m = jnp.max(logits, axis=1, keepdims=True)
        lse = jnp.log(jnp.sum(jnp.exp(logits - m), axis=1, keepdims=True)) + m
        o_ref[...] = logits - lse

    return pl.pallas_call(
        body,
        out_shape=jax.ShapeDtypeStruct((N, D), jnp.float32),
    )(q, denp, wf1, bf1, wf2p, bf2p)


def kernel(x, edge_index, subgraph_node_idx,
           W_root0, W_nbr0, b0, gamma0, beta0,
           W_root1, W_nbr1, b1, gamma1, beta1,
           W_root2, W_nbr2, b2, gamma2, beta2,
           Wf1, bf1, Wf2, bf2):
    pad_e = E_PAD - E
    src_p = jnp.concatenate([edge_index[0], jnp.zeros((pad_e,), jnp.int32)])
    dst_p = jnp.concatenate([edge_index[1], jnp.full((pad_e,), N, jnp.int32)])
    ei4 = jnp.stack([src_p, dst_p]).reshape(
        2, NW, N_CHUNKS, CH).transpose(1, 2, 0, 3)
    dst3 = dst_p.reshape(NW, N_CHUNKS, CH)
    sgi_pad = jnp.concatenate(
        [subgraph_node_idx, jnp.full((R_PAD - N,), N, jnp.int32)])
    src_lin = jnp.concatenate(
        [jnp.arange(N, dtype=jnp.int32), jnp.zeros((R_PAD - N,), jnp.int32)])
    ri4 = jnp.stack([src_lin, sgi_pad]).reshape(
        2, NW, R_CHUNKS, CH).transpose(1, 2, 0, 3)
    sgi3 = sgi_pad.reshape(NW, R_CHUNKS, CH)
    zerosD = jnp.zeros((N_TAB, D), jnp.float32)

    onesD = jnp.ones((CH, D), jnp.float32)
    cnt_p, den_p = _sc_counts(dst3, sgi3, zerosD, onesD)

    roots = [W_root0, W_root1, W_root2]
    nbrs = [W_nbr0, W_nbr1, W_nbr2]
    bs = [b0.reshape(1, D), b1.reshape(1, D), b2.reshape(1, D)]
    gs = [gamma0.reshape(1, D), gamma1.reshape(1, D), gamma2.reshape(1, D)]
    bes = [beta0.reshape(1, D), beta1.reshape(1, D), beta2.reshape(1, D)]

    y = _tc_mm(x, nbrs[0])
    h = x
    for i in range(3):
        p = _sc_scatter_edges(y, ei4, zerosD)
        if i < 2:
            h, y = _tc_fuse(h, p, cnt_p, roots[i], bs[i], gs[i], bes[i],
                            nbrs[i + 1])
        else:
            h3 = _tc_fin(h, p, cnt_p, roots[i], bs[i], gs[i], bes[i])

    q = _sc_scatter_read(h3, ri4, zerosD)

    wf2p = jnp.zeros((2 * D, D), jnp.float32).at[:, :T].set(Wf2)
    bf2p = jnp.zeros((1, D), jnp.float32).at[0, :T].set(bf2)
    out = _tc_head(q, den_p, Wf1, bf1.reshape(1, 2 * D), wf2p, bf2p)
    return out[:, :T]
